# baseline (speedup 1.0000x reference)
import functools

import jax
import jax.numpy as jnp
from jax import lax
from jax.experimental import pallas as pl
from jax.experimental.pallas import tpu as pltpu
from jax.experimental.pallas import tpu_sc as plsc

NUM_CORES = 2
NUM_SUBCORES = 16
NUM_WORKERS = NUM_CORES * NUM_SUBCORES
CHUNK = 16
NSLOT = 3
GROUP = 4 * CHUNK


def _make_gather(d_model: int, total: int):
    b_per_w = total // NUM_WORKERS
    n_groups = b_per_w // GROUP
    mesh = plsc.VectorSubcoreMesh(
        core_axis_name="c", subcore_axis_name="s", num_cores=NUM_CORES
    )

    @functools.partial(
        pl.kernel,
        out_type=jax.ShapeDtypeStruct((total, d_model), jnp.float32),
        mesh=mesh,
        scratch_types=[
            pltpu.VMEM((b_per_w,), jnp.int32),
            [pltpu.VMEM((CHUNK, d_model), jnp.float32) for _ in range(NSLOT)],
            pltpu.VMEM((CHUNK, d_model), jnp.float32),
            pltpu.VMEM_SHARED((NUM_SUBCORES, NSLOT, CHUNK, d_model), jnp.float32),
            [pltpu.SemaphoreType.DMA for _ in range(NSLOT)],
            [pltpu.SemaphoreType.DMA for _ in range(NSLOT)],
            [pltpu.SemaphoreType.DMA for _ in range(NSLOT)],
            pltpu.SemaphoreType.DMA,
            pltpu.SemaphoreType.DMA,
        ],
    )
    def sc_gather(table_hbm, idx_hbm, out_hbm, idx_v, sbufs, dbuf, shared,
                  gsems, csems, osems, dgsem, dosem):
        wid = lax.axis_index("s") * NUM_CORES + lax.axis_index("c")
        sid = lax.axis_index("s")
        base = wid * b_per_w
        pltpu.sync_copy(idx_hbm.at[pl.ds(base, b_per_w)], idx_v)

        def start_gather(row_off, j):
            idx_slice = idx_v.at[pl.ds(row_off, CHUNK)]
            pltpu.async_copy(table_hbm.at[idx_slice], sbufs[j], gsems[j])

        def wait_gather(j):
            idx_slice = idx_v.at[pl.ds(0, CHUNK)]
            pltpu.make_async_copy(table_hbm.at[idx_slice], sbufs[j],
                                  gsems[j]).wait()

        def start_copy(j):
            pltpu.async_copy(sbufs[j], shared.at[sid, j], csems[j])

        def wait_copy(j):
            pltpu.make_async_copy(sbufs[j], shared.at[sid, j], csems[j]).wait()

        def start_out(row_off, j):
            pltpu.async_copy(shared.at[sid, j],
                             out_hbm.at[pl.ds(base + row_off, CHUNK)], osems[j])

        def wait_out(j):
            pltpu.make_async_copy(shared.at[sid, j],
                                  out_hbm.at[pl.ds(base, CHUNK)], osems[j]).wait()

        def start_dgather(row_off):
            idx_slice = idx_v.at[pl.ds(row_off, CHUNK)]
            pltpu.async_copy(table_hbm.at[idx_slice], dbuf, dgsem)

        def wait_dgather():
            idx_slice = idx_v.at[pl.ds(0, CHUNK)]
            pltpu.make_async_copy(table_hbm.at[idx_slice], dbuf, dgsem).wait()

        def start_dout(row_off):
            pltpu.async_copy(dbuf, out_hbm.at[pl.ds(base + row_off, CHUNK)],
                             dosem)

        def wait_dout():
            pltpu.make_async_copy(dbuf, out_hbm.at[pl.ds(base, CHUNK)],
                                  dosem).wait()

        def spmem_chunks(off, first):
            for j in range(NSLOT):
                wait_gather(j)
                if not first:
                    wait_out(j)
                start_copy(j)
                wait_copy(j)
                start_out(off + j * CHUNK, j)
                start_gather(off + GROUP + j * CHUNK, j)

        for j in range(NSLOT):
            start_gather(j * CHUNK, j)
        start_dgather(NSLOT * CHUNK)

        wait_dgather()
        start_dout(NSLOT * CHUNK)
        spmem_chunks(0, first=True)
        wait_dout()
        start_dgather(GROUP + NSLOT * CHUNK)

        def body(p, carry):
            off = pl.multiple_of(p * GROUP, GROUP)
            wait_dgather()
            start_dout(off + NSLOT * CHUNK)
            spmem_chunks(off, first=False)
            wait_dout()
            start_dgather(off + GROUP + NSLOT * CHUNK)
            return carry

        lax.fori_loop(1, n_groups - 1, body, 0)

        off = (n_groups - 1) * GROUP
        wait_dgather()
        start_dout(off + NSLOT * CHUNK)
        for j in range(NSLOT):
            wait_gather(j)
            wait_out(j)
            start_copy(j)
            wait_copy(j)
            start_out(off + j * CHUNK, j)
        for j in range(NSLOT):
            wait_out(j)
        wait_dout()

    return sc_gather


def kernel(positions, pe):
    b, s = positions.shape
    n_rows, d_model = pe.shape
    idx = positions.reshape(b * s)
    out = _make_gather(d_model, b * s)(pe, idx)
    return out.reshape(b, s, d_model)
